# baseline (speedup 1.0000x reference)
import functools

import jax
import jax.numpy as jnp
from jax import lax
from jax.experimental import pallas as pl
from jax.experimental.pallas import tpu as pltpu
from jax.experimental.pallas import tpu_sc as plsc

_NC = 2
_NS = 16
_NW = _NC * _NS
_W = 128


def kernel(x, table):
    B, S = x.shape
    V, D = table.shape
    N = B * S
    n_chunk = N // (_NW * _W)
    assert N == _NW * n_chunk * _W

    idx = x.reshape(_NW, n_chunk, _W)
    mesh = plsc.VectorSubcoreMesh(core_axis_name="c", subcore_axis_name="s")

    @functools.partial(
        pl.kernel,
        mesh=mesh,
        compiler_params=pltpu.CompilerParams(use_tc_tiling_on_sc=False),
        out_type=jax.ShapeDtypeStruct((N, D), table.dtype),
        scratch_types=[
            pltpu.VMEM((n_chunk, _W), jnp.int32),
            pltpu.VMEM((_W, D), jnp.float32),
            pltpu.SemaphoreType.DMA,
        ],
    )
    def gather_kernel(table_hbm, idx_hbm, out_hbm, idx_v, rows_v, sem):
        wid = lax.axis_index("s") * _NC + lax.axis_index("c")
        base = wid * (n_chunk * _W)
        pltpu.sync_copy(idx_hbm.at[wid], idx_v)

        @pl.loop(0, n_chunk)
        def _(j):
            pltpu.async_copy(table_hbm.at[idx_v.at[j]], rows_v, sem).wait()
            pltpu.sync_copy(rows_v, out_hbm.at[pl.ds(base + j * _W, _W)])

    out = gather_kernel(table, idx)
    return out.reshape(B, S, D)


# speedup vs baseline: 1.3610x; 1.3610x over previous
import functools

import jax
import jax.numpy as jnp
from jax import lax
from jax.experimental import pallas as pl
from jax.experimental.pallas import tpu as pltpu
from jax.experimental.pallas import tpu_sc as plsc

_NC = 2
_NS = 16
_NW = _NC * _NS
_W = 128
_NBUF = 4


def kernel(x, table):
    B, S = x.shape
    V, D = table.shape
    DP = 128
    N = B * S
    n_chunk = N // (_NW * _W)
    assert N == _NW * n_chunk * _W and n_chunk % _NBUF == 0

    table_p = jnp.pad(table, ((0, 0), (0, DP - D)))
    idx = x.reshape(_NW, n_chunk, _W)
    mesh = plsc.VectorSubcoreMesh(core_axis_name="c", subcore_axis_name="s")

    @functools.partial(
        pl.kernel,
        mesh=mesh,
        out_type=jax.ShapeDtypeStruct((N, DP), table.dtype),
        scratch_types=[
            pltpu.VMEM((n_chunk, _W), jnp.int32),
            pltpu.VMEM((_NBUF, _W, DP), jnp.float32),
            [pltpu.SemaphoreType.DMA] * _NBUF,
            [pltpu.SemaphoreType.DMA] * _NBUF,
        ],
    )
    def gather_kernel(table_hbm, idx_hbm, out_hbm, idx_v, rows_v, gsems, wsems):
        wid = lax.axis_index("s") * _NC + lax.axis_index("c")
        base = wid * (n_chunk * _W)
        pltpu.sync_copy(idx_hbm.at[wid], idx_v)

        def start_gather(j, b):
            pltpu.async_copy(
                table_hbm.at[idx_v.at[j]], rows_v.at[b], gsems[b]
            )

        def start_write(j, b):
            pltpu.async_copy(
                rows_v.at[b], out_hbm.at[pl.ds(base + j * _W, _W)], wsems[b]
            )

        start_gather(0, 0)
        start_gather(1, 1)

        @pl.loop(0, n_chunk, step=_NBUF)
        def _(j0):
            for b in range(_NBUF):
                j = j0 + b
                bg = (b + 2) % _NBUF

                def wait_write(bb=bg):
                    pltpu.make_async_copy(
                        rows_v.at[bb],
                        out_hbm.at[pl.ds(base, _W)],
                        wsems[bb],
                    ).wait()

                if b < 2:
                    @pl.when(j0 > 0)
                    def _():
                        wait_write()
                else:
                    wait_write()

                @pl.when(j + 2 < n_chunk)
                def _():
                    start_gather(j + 2, bg)

                pltpu.make_async_copy(
                    table_hbm.at[idx_v.at[j]], rows_v.at[b], gsems[b]
                ).wait()
                start_write(j, b)

        for b in (_NBUF - 2, _NBUF - 1):
            pltpu.make_async_copy(
                rows_v.at[b], out_hbm.at[pl.ds(base, _W)], wsems[b]
            ).wait()

    out = gather_kernel(table_p, idx)
    return out[:, :D].reshape(B, S, D)


# speedup vs baseline: 1.6660x; 1.2241x over previous
import functools

import jax
import jax.numpy as jnp
from jax import lax
from jax.experimental import pallas as pl
from jax.experimental.pallas import tpu as pltpu
from jax.experimental.pallas import tpu_sc as plsc

_NC = 2
_NS = 16
_NW = _NC * _NS
_W = 128
_NBUF = 4


def _transpose_pad(table_t, V, D, DP):
    X = 4096

    def body(in_ref, out_ref):
        out_ref[:, :D] = in_ref[...].T

    return pl.pallas_call(
        body,
        grid=(pl.cdiv(V, X),),
        in_specs=[pl.BlockSpec((D, X), lambda i: (0, i))],
        out_specs=pl.BlockSpec((X, DP), lambda i: (i, 0)),
        out_shape=jax.ShapeDtypeStruct((V, DP), jnp.float32),
    )(table_t)


def kernel(x, table):
    B, S = x.shape
    V, D = table.shape
    DP = 128
    N = B * S
    n_chunk = N // (_NW * _W)
    assert N == _NW * n_chunk * _W and n_chunk % _NBUF == 0

    table_p = _transpose_pad(table.T, V, D, DP)
    idx = x.reshape(_NW, n_chunk, _W)
    mesh = plsc.VectorSubcoreMesh(core_axis_name="c", subcore_axis_name="s")

    @functools.partial(
        pl.kernel,
        mesh=mesh,
        out_type=jax.ShapeDtypeStruct((N, DP), table.dtype),
        scratch_types=[
            pltpu.VMEM((n_chunk, _W), jnp.int32),
            pltpu.VMEM((_NBUF, _W, DP), jnp.float32),
            [pltpu.SemaphoreType.DMA] * _NBUF,
            [pltpu.SemaphoreType.DMA] * _NBUF,
        ],
    )
    def gather_kernel(table_hbm, idx_hbm, out_hbm, idx_v, rows_v, gsems, wsems):
        wid = lax.axis_index("s") * _NC + lax.axis_index("c")
        base = wid * (n_chunk * _W)
        pltpu.sync_copy(idx_hbm.at[wid], idx_v)

        def start_gather(j, b):
            pltpu.async_copy(
                table_hbm.at[idx_v.at[j]], rows_v.at[b], gsems[b]
            )

        def start_write(j, b):
            pltpu.async_copy(
                rows_v.at[b], out_hbm.at[pl.ds(base + j * _W, _W)], wsems[b]
            )

        start_gather(0, 0)
        start_gather(1, 1)

        @pl.loop(0, n_chunk, step=_NBUF)
        def _(j0):
            for b in range(_NBUF):
                j = j0 + b
                bg = (b + 2) % _NBUF

                def wait_write(bb=bg):
                    pltpu.make_async_copy(
                        rows_v.at[bb],
                        out_hbm.at[pl.ds(base, _W)],
                        wsems[bb],
                    ).wait()

                if b < 2:
                    @pl.when(j0 > 0)
                    def _():
                        wait_write()
                else:
                    wait_write()

                @pl.when(j + 2 < n_chunk)
                def _():
                    start_gather(j + 2, bg)

                pltpu.make_async_copy(
                    table_hbm.at[idx_v.at[j]], rows_v.at[b], gsems[b]
                ).wait()
                start_write(j, b)

        for b in (_NBUF - 2, _NBUF - 1):
            pltpu.make_async_copy(
                rows_v.at[b], out_hbm.at[pl.ds(base, _W)], wsems[b]
            ).wait()

    out = gather_kernel(table_p, idx)
    return out[:, :D].reshape(B, S, D)


# speedup vs baseline: 1.8282x; 1.0974x over previous
import functools

import jax
import jax.numpy as jnp
from jax import lax
from jax.experimental import pallas as pl
from jax.experimental.pallas import tpu as pltpu
from jax.experimental.pallas import tpu_sc as plsc

_NC = 2
_NS = 16
_NW = _NC * _NS
_W = 128
_NBUF = 4


def _transpose_pad(table_t, V, D, DP):
    X = 8192

    def body(in_ref, out_ref):
        out_ref[:, :D] = in_ref[...].T

    return pl.pallas_call(
        body,
        grid=(pl.cdiv(V, X),),
        in_specs=[pl.BlockSpec((D, X), lambda i: (0, i))],
        out_specs=pl.BlockSpec((X, DP), lambda i: (i, 0)),
        out_shape=jax.ShapeDtypeStruct((V, DP), jnp.float32),
        compiler_params=pltpu.CompilerParams(
            dimension_semantics=("parallel",)
        ),
    )(table_t)


def kernel(x, table):
    B, S = x.shape
    V, D = table.shape
    DP = 128
    N = B * S
    n_chunk = N // (_NW * _W)
    assert N == _NW * n_chunk * _W and n_chunk % _NBUF == 0

    table_p = _transpose_pad(table.T, V, D, DP)
    idx = x.reshape(_NW, n_chunk, _W)
    mesh = plsc.VectorSubcoreMesh(core_axis_name="c", subcore_axis_name="s")

    @functools.partial(
        pl.kernel,
        mesh=mesh,
        out_type=jax.ShapeDtypeStruct((N, DP), table.dtype),
        scratch_types=[
            pltpu.VMEM((n_chunk, _W), jnp.int32),
            pltpu.VMEM((_NBUF, _W, DP), jnp.float32),
            [pltpu.SemaphoreType.DMA] * _NBUF,
            [pltpu.SemaphoreType.DMA] * _NBUF,
        ],
    )
    def gather_kernel(table_hbm, idx_hbm, out_hbm, idx_v, rows_v, gsems, wsems):
        wid = lax.axis_index("s") * _NC + lax.axis_index("c")
        base = wid * (n_chunk * _W)
        pltpu.sync_copy(idx_hbm.at[wid], idx_v)

        def start_gather(j, b):
            pltpu.async_copy(
                table_hbm.at[idx_v.at[j]], rows_v.at[b], gsems[b]
            )

        def start_write(j, b):
            pltpu.async_copy(
                rows_v.at[b], out_hbm.at[pl.ds(base + j * _W, _W)], wsems[b]
            )

        start_gather(0, 0)
        start_gather(1, 1)

        @pl.loop(0, n_chunk, step=_NBUF)
        def _(j0):
            for b in range(_NBUF):
                j = j0 + b
                bg = (b + 2) % _NBUF

                def wait_write(bb=bg):
                    pltpu.make_async_copy(
                        rows_v.at[bb],
                        out_hbm.at[pl.ds(base, _W)],
                        wsems[bb],
                    ).wait()

                if b < 2:
                    @pl.when(j0 > 0)
                    def _():
                        wait_write()
                else:
                    wait_write()

                @pl.when(j + 2 < n_chunk)
                def _():
                    start_gather(j + 2, bg)

                pltpu.make_async_copy(
                    table_hbm.at[idx_v.at[j]], rows_v.at[b], gsems[b]
                ).wait()
                start_write(j, b)

        for b in (_NBUF - 2, _NBUF - 1):
            pltpu.make_async_copy(
                rows_v.at[b], out_hbm.at[pl.ds(base, _W)], wsems[b]
            ).wait()

    out = gather_kernel(table_p, idx)
    return out[:, :D].reshape(B, S, D)
